# baseline (speedup 1.0000x reference)
import jax
import jax.numpy as jnp
from jax.experimental import pallas as pl
from jax.experimental.pallas import tpu as pltpu

SEL = 64
GRP = 8


def _scope_kernel(nsel_ref, hid_hbm, cls_ref, tok_ref, idx_ref,
                  hid_ref, cos_ref, sel_ref, cmax_ref, idxr_ref,
                  dma_sem):
    pid = pl.program_id(0)
    copy = pltpu.make_async_copy(
        hid_hbm.at[pl.ds(pid * GRP, GRP)], hid_ref, dma_sem)
    copy.start()
    copy.wait()

    n_tok = hid_ref.shape[1] - 1
    nsel = nsel_ref[0, 0]
    lane_n = jax.lax.broadcasted_iota(jnp.int32, (1, n_tok), 1)
    lane_k = jax.lax.broadcasted_iota(jnp.int32, (1, SEL), 1)
    col_k = jax.lax.broadcasted_iota(jnp.int32, (SEL, 1), 0)

    for g in range(GRP):
        feat = hid_ref[g, 1:, :]
        nrm = jnp.sqrt(jnp.sum(feat * feat, axis=1, keepdims=True))
        normf = feat / nrm
        cos_ref[g] = jax.lax.dot_general(
            normf, normf, (((1,), (1,)), ((), ())),
            preferred_element_type=jnp.float32)

    sel_ref[...] = jnp.zeros(sel_ref.shape, dtype=jnp.float32)
    cmax_ref[...] = jnp.zeros(cmax_ref.shape, dtype=jnp.float32)
    idxr_ref[...] = jnp.zeros(idxr_ref.shape, dtype=jnp.int32)

    def body(t, _):
        for u in range(2):
            i = t * 2 + u
            active = i < nsel
            for g in range(GRP):
                selected = sel_ref[g]
                cur_max = cmax_ref[g]
                gsum = jnp.sum(jnp.maximum(cos_ref[g] - cur_max, 0.0),
                               axis=0, keepdims=True)
                gsum = gsum * cls_ref[g]
                gsum = jnp.where(selected > 0.0, -jnp.inf, gsum)
                m = jnp.max(gsum, keepdims=True)
                bv = jnp.min(jnp.where(gsum == m, lane_n, n_tok),
                             keepdims=True)
                best = bv[0, 0]
                sel_ref[g] = jnp.where(
                    active & (lane_n == bv), 1.0, selected)
                idxr_ref[g] = jnp.where(
                    active & (lane_k == i), bv, idxr_ref[g])
                best_col = jnp.transpose(cos_ref[g, pl.ds(best, 1), :])
                best_col = jnp.where(active, best_col, -jnp.inf)
                cmax_ref[g] = jnp.maximum(cur_max, best_col)
        return 0

    jax.lax.fori_loop(0, SEL // 2, body, 0)

    for g in range(GRP):
        idx_row = idxr_ref[g]
        idx_col = jnp.transpose(idx_row)
        idx_ref[g, 0] = idx_row[0] + 1

        cmp = (idx_col < idx_row) | ((idx_col == idx_row) & (col_k < lane_k))
        rank_row = jnp.sum(cmp.astype(jnp.int32), axis=0, keepdims=True)
        perm = (rank_row == col_k)
        sorted_col = jnp.sum(jnp.where(perm, idx_row, 0),
                             axis=1, keepdims=True)

        onehot = (sorted_col == lane_n).astype(jnp.float32)
        tok_ref[g] = jax.lax.dot_general(
            onehot, hid_ref[g, 1:, :], (((1,), (0,)), ((), ())),
            preferred_element_type=jnp.float32,
            precision=jax.lax.Precision.HIGHEST)


def kernel(hidden_states, cls_attn, dominant_num):
    B, N1, D = hidden_states.shape
    N = N1 - 1
    nsel = jnp.asarray(dominant_num, jnp.int32).reshape(1, 1)
    cls_row = cls_attn[:, None, :]
    tok, idx = pl.pallas_call(
        _scope_kernel,
        grid=(B // GRP,),
        in_specs=[
            pl.BlockSpec(memory_space=pltpu.SMEM),
            pl.BlockSpec(memory_space=pl.ANY),
            pl.BlockSpec((GRP, 1, N), lambda b: (b, 0, 0)),
        ],
        out_specs=[
            pl.BlockSpec((GRP, SEL, D), lambda b: (b, 0, 0)),
            pl.BlockSpec((GRP, 1, SEL), lambda b: (b, 0, 0)),
        ],
        out_shape=[
            jax.ShapeDtypeStruct((B, SEL, D), jnp.float32),
            jax.ShapeDtypeStruct((B, 1, SEL), jnp.int32),
        ],
        scratch_shapes=[
            pltpu.VMEM((GRP, N1, D), jnp.float32),
            pltpu.VMEM((GRP, N, N), jnp.float32),
            pltpu.VMEM((GRP, 1, N), jnp.float32),
            pltpu.VMEM((GRP, N, 1), jnp.float32),
            pltpu.VMEM((GRP, 1, SEL), jnp.int32),
            pltpu.SemaphoreType.DMA,
        ],
        compiler_params=pltpu.CompilerParams(
            dimension_semantics=("parallel",)),
    )(nsel, hidden_states, cls_row)
    return tok, idx.reshape(B, SEL)


# speedup vs baseline: 1.1418x; 1.1418x over previous
import jax
import jax.numpy as jnp
from jax.experimental import pallas as pl
from jax.experimental.pallas import tpu as pltpu

SEL = 64
GRP = 8


def _scope_kernel(nsel_ref, hid_hbm, cls_ref, tok_ref, idx_ref,
                  hid_ref, cos_ref, sel_ref, cmax_ref, idxr_ref,
                  dma_sem):
    pid = pl.program_id(0)
    copy = pltpu.make_async_copy(
        hid_hbm.at[pl.ds(pid * GRP, GRP)], hid_ref, dma_sem)
    copy.start()
    copy.wait()

    n_tok = hid_ref.shape[1] - 1
    nsel = nsel_ref[0, 0]
    lane_n = jax.lax.broadcasted_iota(jnp.int32, (1, n_tok), 1)
    lane_k = jax.lax.broadcasted_iota(jnp.int32, (1, SEL), 1)
    col_k = jax.lax.broadcasted_iota(jnp.int32, (SEL, 1), 0)

    for g in range(GRP):
        feat = hid_ref[g, 1:, :]
        nrm = jnp.sqrt(jnp.sum(feat * feat, axis=1, keepdims=True))
        normf = feat / nrm
        cos_ref[g] = jax.lax.dot_general(
            normf, normf, (((1,), (1,)), ((), ())),
            preferred_element_type=jnp.float32)

    sel_ref[...] = jnp.zeros(sel_ref.shape, dtype=jnp.float32)
    cmax_ref[...] = jnp.zeros(cmax_ref.shape, dtype=jnp.float32)
    idxr_ref[...] = jnp.zeros(idxr_ref.shape, dtype=jnp.int32)

    def body(i, _):
        for g in range(GRP):
            selected = sel_ref[g]
            cur_max = cmax_ref[g]
            gsum = jnp.sum(jnp.maximum(cos_ref[g] - cur_max, 0.0),
                           axis=0, keepdims=True)
            gsum = gsum * cls_ref[g]
            gsum = jnp.where(selected > 0.0, -jnp.inf, gsum)
            m = jnp.max(gsum, keepdims=True)
            bv = jnp.min(jnp.where(gsum == m, lane_n, n_tok),
                         keepdims=True)
            best = bv[0, 0]
            sel_ref[g] = jnp.where(lane_n == bv, 1.0, selected)
            idxr_ref[g] = jnp.where(lane_k == i, bv, idxr_ref[g])
            best_col = jnp.transpose(cos_ref[g, pl.ds(best, 1), :])
            cmax_ref[g] = jnp.maximum(cur_max, best_col)
        return 0

    jax.lax.fori_loop(0, jnp.minimum(nsel, SEL), body, 0)

    for g in range(GRP):
        idx_row = idxr_ref[g]
        idx_col = jnp.transpose(idx_row)
        idx_ref[g, 0] = idx_row[0] + 1

        cmp = (idx_col < idx_row) | ((idx_col == idx_row) & (col_k < lane_k))
        rank_row = jnp.sum(cmp.astype(jnp.int32), axis=0, keepdims=True)
        perm = (rank_row == col_k)
        sorted_col = jnp.sum(jnp.where(perm, idx_row, 0),
                             axis=1, keepdims=True)

        onehot = (sorted_col == lane_n).astype(jnp.float32)
        tok_ref[g] = jax.lax.dot_general(
            onehot, hid_ref[g, 1:, :], (((1,), (0,)), ((), ())),
            preferred_element_type=jnp.float32,
            precision=jax.lax.Precision.HIGHEST)


def kernel(hidden_states, cls_attn, dominant_num):
    B, N1, D = hidden_states.shape
    N = N1 - 1
    nsel = jnp.asarray(dominant_num, jnp.int32).reshape(1, 1)
    cls_row = cls_attn[:, None, :]
    tok, idx = pl.pallas_call(
        _scope_kernel,
        grid=(B // GRP,),
        in_specs=[
            pl.BlockSpec(memory_space=pltpu.SMEM),
            pl.BlockSpec(memory_space=pl.ANY),
            pl.BlockSpec((GRP, 1, N), lambda b: (b, 0, 0)),
        ],
        out_specs=[
            pl.BlockSpec((GRP, SEL, D), lambda b: (b, 0, 0)),
            pl.BlockSpec((GRP, 1, SEL), lambda b: (b, 0, 0)),
        ],
        out_shape=[
            jax.ShapeDtypeStruct((B, SEL, D), jnp.float32),
            jax.ShapeDtypeStruct((B, 1, SEL), jnp.int32),
        ],
        scratch_shapes=[
            pltpu.VMEM((GRP, N1, D), jnp.float32),
            pltpu.VMEM((GRP, N, N), jnp.float32),
            pltpu.VMEM((GRP, 1, N), jnp.float32),
            pltpu.VMEM((GRP, N, 1), jnp.float32),
            pltpu.VMEM((GRP, 1, SEL), jnp.int32),
            pltpu.SemaphoreType.DMA,
        ],
        compiler_params=pltpu.CompilerParams(
            dimension_semantics=("parallel",)),
    )(nsel, hidden_states, cls_row)
    return tok, idx.reshape(B, SEL)
